# baseline (speedup 1.0000x reference)
import dataclasses
import functools

import jax
import jax.numpy as jnp
from jax import lax
from jax.experimental import pallas as pl
from jax.experimental.pallas import tpu as pltpu
from jax.experimental.pallas import tpu_sc as plsc

_NUM_EXPERTS = 8
_TOPK = 2
_BLOCK = 8192
_NUM_CORES = 2
_NUM_SUBCORES = 16
_NW = _NUM_CORES * _NUM_SUBCORES
_LANES = 16


def _probs_block(x_ref, wt_ref, b_ref, p_out):
    logits = jax.lax.dot_general(
        x_ref[...], wt_ref[...], (((1,), (0,)), ((), ())),
        preferred_element_type=jnp.float32,
    ) + b_ref[...]
    p_out[...] = jax.nn.sigmoid(logits)


def _tc_probs(hidden_states, wt, b2):
    n_tokens, hidden = hidden_states.shape
    n_exp = wt.shape[1]
    return pl.pallas_call(
        _probs_block,
        grid=(n_tokens // _BLOCK,),
        in_specs=[
            pl.BlockSpec((_BLOCK, hidden), lambda i: (i, 0)),
            pl.BlockSpec((hidden, n_exp), lambda i: (0, 0)),
            pl.BlockSpec((1, n_exp), lambda i: (0, 0)),
        ],
        out_specs=pl.BlockSpec((_BLOCK, n_exp), lambda i: (i, 0)),
        out_shape=jax.ShapeDtypeStruct((n_tokens, n_exp), jnp.float32),
    )(hidden_states, wt, b2)


def _sc_route(probs_flat, n_tokens):
    tok_per_w = n_tokens // _NW
    chunk = tok_per_w * _NUM_EXPERTS
    n_groups = tok_per_w // _LANES
    mesh = plsc.VectorSubcoreMesh(core_axis_name="c", subcore_axis_name="s")
    cp = pltpu.CompilerParams()
    if "needs_layout_passes" in pltpu.CompilerParams.__dataclass_fields__:
        cp = dataclasses.replace(cp, needs_layout_passes=False)

    @functools.partial(
        pl.kernel,
        mesh=mesh,
        compiler_params=cp,
        out_type=[
            jax.ShapeDtypeStruct((n_tokens * _TOPK,), jnp.float32),
            jax.ShapeDtypeStruct((n_tokens * _TOPK,), jnp.int32),
        ],
        scratch_types=[
            pltpu.VMEM((chunk,), jnp.float32),
            pltpu.VMEM((tok_per_w * _TOPK,), jnp.float32),
            pltpu.VMEM((tok_per_w * _TOPK,), jnp.int32),
        ],
    )
    def route(p_hbm, w_hbm, i_hbm, p_v, w_v, i_v):
        wid = lax.axis_index("s") * _NUM_CORES + lax.axis_index("c")
        base = wid * tok_per_w
        pltpu.sync_copy(p_hbm.at[pl.ds(base * _NUM_EXPERTS, chunk)], p_v)

        lane = lax.iota(jnp.int32, _LANES)
        gidx0 = lane * _NUM_EXPERTS
        widx0 = lane * _TOPK

        @pl.loop(0, n_groups)
        def _(g):
            gbase = gidx0 + g * (_LANES * _NUM_EXPERTS)
            v = plsc.load_gather(p_v, [gbase])
            m1 = v
            i1 = jnp.zeros((_LANES,), jnp.int32)
            m2 = jnp.full((_LANES,), -1.0, jnp.float32)
            i2 = jnp.zeros((_LANES,), jnp.int32)
            for e in range(1, _NUM_EXPERTS):
                v = plsc.load_gather(p_v, [gbase + e])
                ecst = jnp.full((_LANES,), e, jnp.int32)
                gt1 = v > m1
                gt2 = v > m2
                m2n = jnp.where(gt1, m1, jnp.where(gt2, v, m2))
                i2n = jnp.where(gt1, i1, jnp.where(gt2, ecst, i2))
                m1 = jnp.where(gt1, v, m1)
                i1 = jnp.where(gt1, ecst, i1)
                m2 = m2n
                i2 = i2n
            s = m1 + m2
            wi = widx0 + g * (_LANES * _TOPK)
            plsc.store_scatter(w_v, [wi], m1 / s)
            plsc.store_scatter(w_v, [wi + 1], m2 / s)
            plsc.store_scatter(i_v, [wi], i1)
            plsc.store_scatter(i_v, [wi + 1], i2)

        pltpu.sync_copy(w_v, w_hbm.at[pl.ds(base * _TOPK, tok_per_w * _TOPK)])
        pltpu.sync_copy(i_v, i_hbm.at[pl.ds(base * _TOPK, tok_per_w * _TOPK)])

    return route(probs_flat)


def kernel(hidden_states, W, b):
    n_tokens, hidden = hidden_states.shape
    n_exp = W.shape[0]
    wt = W.T
    b2 = b.reshape(1, n_exp)
    probs = _tc_probs(hidden_states, wt, b2)
    rw, ri = jax.lax.top_k(probs, _TOPK)
    rw = rw / jnp.sum(rw, axis=-1, keepdims=True)
    return (rw, ri, probs)


# speedup vs baseline: 1.3308x; 1.3308x over previous
import dataclasses
import functools

import jax
import jax.numpy as jnp
from jax import lax
from jax.experimental import pallas as pl
from jax.experimental.pallas import tpu as pltpu
from jax.experimental.pallas import tpu_sc as plsc

_NUM_EXPERTS = 8
_TOPK = 2
_BLOCK = 4096
_NUM_CORES = 2
_NUM_SUBCORES = 16
_NW = _NUM_CORES * _NUM_SUBCORES
_LANES = 16


def _probs_block(x_ref, wt_ref, b_ref, p_out):
    logits = jax.lax.dot_general(
        x_ref[...], wt_ref[...], (((1,), (0,)), ((), ())),
        preferred_element_type=jnp.float32,
    ) + b_ref[...]
    p_out[...] = jax.nn.sigmoid(logits).T


def _tc_probs(hidden_states, wt, b2):
    n_tokens, hidden = hidden_states.shape
    n_exp = wt.shape[1]
    return pl.pallas_call(
        _probs_block,
        grid=(n_tokens // _BLOCK,),
        in_specs=[
            pl.BlockSpec((_BLOCK, hidden), lambda i: (i, 0)),
            pl.BlockSpec((hidden, n_exp), lambda i: (0, 0)),
            pl.BlockSpec((1, n_exp), lambda i: (0, 0)),
        ],
        out_specs=pl.BlockSpec((n_exp, _BLOCK), lambda i: (0, i)),
        out_shape=jax.ShapeDtypeStruct((n_exp, n_tokens), jnp.float32),
    )(hidden_states, wt, b2)


def _sc_route(probs_flat, n_tokens):
    tok_per_w = n_tokens // _NW
    chunk = tok_per_w * _NUM_EXPERTS
    n_groups = tok_per_w // _LANES
    mesh = plsc.VectorSubcoreMesh(core_axis_name="c", subcore_axis_name="s")
    cp = pltpu.CompilerParams()
    if "needs_layout_passes" in pltpu.CompilerParams.__dataclass_fields__:
        cp = dataclasses.replace(cp, needs_layout_passes=False)

    @functools.partial(
        pl.kernel,
        mesh=mesh,
        compiler_params=cp,
        out_type=[
            jax.ShapeDtypeStruct((n_tokens * _TOPK,), jnp.float32),
            jax.ShapeDtypeStruct((n_tokens * _TOPK,), jnp.int32),
        ],
        scratch_types=[
            pltpu.VMEM((chunk,), jnp.float32),
            pltpu.VMEM((tok_per_w * _TOPK,), jnp.float32),
            pltpu.VMEM((tok_per_w * _TOPK,), jnp.int32),
        ],
    )
    def route(p_hbm, w_hbm, i_hbm, p_v, w_v, i_v):
        wid = lax.axis_index("s") * _NUM_CORES + lax.axis_index("c")
        base = wid * tok_per_w
        pltpu.sync_copy(p_hbm.at[pl.ds(base * _NUM_EXPERTS, chunk)], p_v)

        lane = lax.iota(jnp.int32, _LANES)
        gidx0 = lane * _NUM_EXPERTS
        widx0 = lane * _TOPK

        @pl.loop(0, n_groups)
        def _(g):
            gbase = gidx0 + g * (_LANES * _NUM_EXPERTS)
            v = plsc.load_gather(p_v, [gbase])
            m1 = v
            i1 = jnp.zeros((_LANES,), jnp.int32)
            m2 = jnp.full((_LANES,), -1.0, jnp.float32)
            i2 = jnp.zeros((_LANES,), jnp.int32)
            for e in range(1, _NUM_EXPERTS):
                v = plsc.load_gather(p_v, [gbase + e])
                ecst = jnp.full((_LANES,), e, jnp.int32)
                gt1 = v > m1
                gt2 = v > m2
                m2n = jnp.where(gt1, m1, jnp.where(gt2, v, m2))
                i2n = jnp.where(gt1, i1, jnp.where(gt2, ecst, i2))
                m1 = jnp.where(gt1, v, m1)
                i1 = jnp.where(gt1, ecst, i1)
                m2 = m2n
                i2 = i2n
            s = m1 + m2
            wi = widx0 + g * (_LANES * _TOPK)
            plsc.store_scatter(w_v, [wi], m1 / s)
            plsc.store_scatter(w_v, [wi + 1], m2 / s)
            plsc.store_scatter(i_v, [wi], i1)
            plsc.store_scatter(i_v, [wi + 1], i2)

        pltpu.sync_copy(w_v, w_hbm.at[pl.ds(base * _TOPK, tok_per_w * _TOPK)])
        pltpu.sync_copy(i_v, i_hbm.at[pl.ds(base * _TOPK, tok_per_w * _TOPK)])

    return route(probs_flat)


def kernel(hidden_states, W, b):
    n_tokens, hidden = hidden_states.shape
    n_exp = W.shape[0]
    wt = W.T
    b2 = b.reshape(1, n_exp)
    probs_t = _tc_probs(hidden_states, wt, b2)
    probs = probs_t.T
    rw, ri = jax.lax.top_k(probs, _TOPK)
    rw = rw / jnp.sum(rw, axis=-1, keepdims=True)
    return (rw, ri, probs)
